# baseline (speedup 1.0000x reference)
import jax
import jax.numpy as jnp
from jax import lax
from jax.experimental import pallas as pl
from jax.experimental.pallas import tpu as pltpu
from jax.experimental.pallas import tpu_sc as plsc

MIN_WINDOW_SIZE = 33
MAX_WINDOW_SIZE = 99

_N = 5000
_NPAD = 5120
_NC, _NS = 2, 16
_NW = _NC * _NS
_PER_W = _NPAD // _NW
_GROUPS = _PER_W // 16
_ROWW = _NPAD // 4
_NBUF = 2
_NP = 26 + 4 * _NBUF


def _half_kernel(boxes_ref, half_ref):
    wh = boxes_ref[:, 2:4]
    mx = jnp.max(wh, axis=1)
    mn = jnp.min(wh, axis=1)
    scale = jnp.sqrt(mx / mn)
    window = (MIN_WINDOW_SIZE * scale).astype(jnp.int32)
    window = jnp.clip(window, MIN_WINDOW_SIZE, MAX_WINDOW_SIZE)
    half_ref[0:1, :_N] = (window // 2).reshape(1, _N)
    half_ref[0:1, _N:] = jnp.zeros((1, _NPAD - _N), jnp.int32)


def _compute_half(boxes):
    return pl.pallas_call(
        _half_kernel,
        out_shape=jax.ShapeDtypeStruct((1, _NPAD), jnp.int32),
    )(boxes)


def _sc_band(half_hbm, out_hbm, half_v, buf, sem0, sem1):
    cid = lax.axis_index("c")
    sid = lax.axis_index("s")
    wid = sid * _NC + cid
    base = wid * _PER_W
    pltpu.sync_copy(half_hbm.at[0], half_v)

    def zbody(k, _):
        for u in range(16):
            buf[pl.ds((k * 16 + u) * 16, 16)] = jnp.zeros((16,), jnp.int32)
        return 0
    lax.fori_loop(0, _NBUF * 16 * _ROWW // 256, zbody, 0)

    lane = lax.iota(jnp.int32, 16)
    sems = (sem0, sem1)
    handles = [None] * _GROUPS

    for g in range(_GROUPS):
        slot = g % _NBUF
        if g >= _NBUF:
            handles[g - _NBUF].wait()

        hv = half_v[pl.ds(base + g * 16, 16)]
        rows = base + g * 16 + lane
        s0 = jnp.maximum(((rows - 49) >> 2) - 4 * _NBUF, 0)
        lanebase = (slot * 16 + lane) * _ROWW

        def pbody(p, _, s0=s0, rows=rows, hv=hv, lanebase=lanebase):
            wp = jnp.minimum(s0 + p, _ROWW - 1)
            b = wp << 2
            acc = jnp.zeros((16,), jnp.int32)
            for m in range(4):
                inb = jnp.abs(b + m - rows) <= hv
                acc = acc + jnp.where(inb, jnp.int32(1 << (8 * m)),
                                      jnp.int32(0))
            plsc.store_scatter(buf, [lanebase + wp], acc)
            return 0
        lax.fori_loop(0, _NP, pbody, 0)

        gidx = wid * _GROUPS + g
        src = buf.at[pl.ds(slot * 16 * _ROWW, 16 * _ROWW)]
        cp = pltpu.make_async_copy(src, out_hbm.at[gidx], sems[slot])
        cp.start()
        handles[g] = cp

    for g in range(_GROUPS - _NBUF, _GROUPS):
        handles[g].wait()


@jax.jit
def kernel(boxes):
    half = _compute_half(boxes)
    mesh = plsc.VectorSubcoreMesh(core_axis_name="c", subcore_axis_name="s")
    sc = pl.kernel(
        _sc_band,
        out_type=jax.ShapeDtypeStruct((_NPAD // 16, 16 * _ROWW), jnp.int32),
        mesh=mesh,
        compiler_params=pltpu.CompilerParams(
            needs_layout_passes=False, use_tc_tiling_on_sc=False),
        scratch_types=[
            pltpu.VMEM((_NPAD,), jnp.int32),
            pltpu.VMEM((_NBUF * 16 * _ROWW,), jnp.int32),
            pltpu.SemaphoreType.DMA,
            pltpu.SemaphoreType.DMA,
        ],
    )
    words = sc(half)
    mask_bytes = lax.bitcast_convert_type(words, jnp.uint8)
    mask_bytes = mask_bytes.reshape(_NPAD, _NPAD)[:_N, :_N]
    return mask_bytes.astype(jnp.bool_)


# speedup vs baseline: 15.3041x; 15.3041x over previous
import jax
import jax.numpy as jnp
from jax import lax
from jax.experimental import pallas as pl
from jax.experimental.pallas import tpu as pltpu
from jax.experimental.pallas import tpu_sc as plsc

MIN_WINDOW_SIZE = 33
MAX_WINDOW_SIZE = 99

_N = 5000
_NPAD = 5120
_NC, _NS = 2, 16
_NW = _NC * _NS
_PER_W = _NPAD // _NW
_GROUPS = _PER_W // 16
_ROWW = _NPAD // 4
_NBUF = 2
_NP = 26 + 4 * _NBUF


def _half_kernel(boxes_ref, half_ref):
    wh = boxes_ref[:, 2:4]
    mx = jnp.max(wh, axis=1)
    mn = jnp.min(wh, axis=1)
    scale = jnp.sqrt(mx / mn)
    window = (MIN_WINDOW_SIZE * scale).astype(jnp.int32)
    window = jnp.clip(window, MIN_WINDOW_SIZE, MAX_WINDOW_SIZE)
    half_ref[0:1, :_N] = (window // 2).reshape(1, _N)
    half_ref[0:1, _N:] = jnp.zeros((1, _NPAD - _N), jnp.int32)


def _compute_half(boxes):
    return pl.pallas_call(
        _half_kernel,
        out_shape=jax.ShapeDtypeStruct((1, _NPAD), jnp.int32),
    )(boxes)


def _sc_band(half_hbm, out_hbm, half_v, buf, sem0, sem1):
    cid = lax.axis_index("c")
    sid = lax.axis_index("s")
    wid = sid * _NC + cid
    base = wid * _PER_W
    pltpu.sync_copy(half_hbm.at[0], half_v)

    def zbody(k, _):
        for slot in range(_NBUF):
            for r in range(16):
                buf[slot, r, pl.ds(k * 16, 16)] = jnp.zeros((16,), jnp.int32)
        return 0
    lax.fori_loop(0, _ROWW // 16, zbody, 0)

    lane = lax.iota(jnp.int32, 16)
    sems = (sem0, sem1)
    handles = [None] * _GROUPS

    for g in range(_GROUPS):
        slot = g % _NBUF
        if g >= _NBUF:
            handles[g - _NBUF].wait()

        hv = half_v[pl.ds(base + g * 16, 16)]
        rows = base + g * 16 + lane
        s0 = jnp.maximum(((rows - 49) >> 2) - 4 * _NBUF, 0)
        slot_v = jnp.full((16,), slot, jnp.int32)

        def pbody(p, _, s0=s0, rows=rows, hv=hv, slot_v=slot_v):
            wp = jnp.minimum(s0 + p, _ROWW - 1)
            b = wp << 2
            acc = jnp.zeros((16,), jnp.int32)
            for m in range(4):
                inb = jnp.abs(b + m - rows) <= hv
                acc = acc + jnp.where(inb, jnp.int32(1 << (8 * m)),
                                      jnp.int32(0))
            plsc.store_scatter(buf, [slot_v, lane, wp], acc)
            return 0
        lax.fori_loop(0, _NP, pbody, 0)

        gidx = wid * _GROUPS + g
        cp = pltpu.make_async_copy(
            buf.at[slot], out_hbm.at[pl.ds(gidx * 16, 16)], sems[slot])
        cp.start()
        handles[g] = cp

    for g in range(_GROUPS - _NBUF, _GROUPS):
        handles[g].wait()


@jax.jit
def kernel(boxes):
    half = _compute_half(boxes)
    mesh = plsc.VectorSubcoreMesh(core_axis_name="c", subcore_axis_name="s")
    sc = pl.kernel(
        _sc_band,
        out_type=jax.ShapeDtypeStruct((_NPAD, _ROWW), jnp.int32),
        mesh=mesh,
        compiler_params=pltpu.CompilerParams(
            needs_layout_passes=False, use_tc_tiling_on_sc=False),
        scratch_types=[
            pltpu.VMEM((_NPAD,), jnp.int32),
            pltpu.VMEM((_NBUF, 16, _ROWW), jnp.int32),
            pltpu.SemaphoreType.DMA,
            pltpu.SemaphoreType.DMA,
        ],
    )
    words = sc(half)
    mask_bytes = lax.bitcast_convert_type(words, jnp.uint8)
    mask_bytes = mask_bytes.reshape(_NPAD, _NPAD)[:_N, :_N]
    return mask_bytes.astype(jnp.bool_)


# speedup vs baseline: 75.1916x; 4.9132x over previous
import jax
import jax.numpy as jnp
from jax.experimental import pallas as pl

MIN_WINDOW_SIZE = 33
MAX_WINDOW_SIZE = 99

_BR = 512
_BW = 768


def _mask_kernel(boxes_ref, out_ref):
    r0 = pl.program_id(0) * _BR
    n = out_ref.shape[1]

    wh = boxes_ref[:, 2:4]
    mx = jnp.max(wh, axis=1)
    mn = jnp.min(wh, axis=1)
    scale = jnp.sqrt(mx / mn)
    window = (MIN_WINDOW_SIZE * scale).astype(jnp.int32)
    window = jnp.clip(window, MIN_WINDOW_SIZE, MAX_WINDOW_SIZE)
    half = (window // 2).astype(jnp.int16)
    h = half[:, None]

    out_ref[...] = jnp.zeros(out_ref.shape, jnp.bool_)

    ncap = ((n - _BW) // 128) * 128
    start = pl.multiple_of(jnp.clip(r0 - 128, 0, ncap), 128)
    i = r0.astype(jnp.int16) + jax.lax.broadcasted_iota(
        jnp.int16, (_BR, _BW), 0)
    j = start.astype(jnp.int16) + jax.lax.broadcasted_iota(
        jnp.int16, (_BR, _BW), 1)
    out_ref[:, pl.ds(start, _BW)] = jnp.abs(j - i) <= h

    ntail = n - (ncap + _BW)
    if ntail > 0:
        jt = jnp.int16(ncap + _BW) + jax.lax.broadcasted_iota(
            jnp.int16, (_BR, ntail), 1)
        it = r0.astype(jnp.int16) + jax.lax.broadcasted_iota(
            jnp.int16, (_BR, ntail), 0)
        out_ref[:, ncap + _BW:] = jnp.abs(jt - it) <= h


@jax.jit
def kernel(boxes):
    n = boxes.shape[0]
    grid = (pl.cdiv(n, _BR),)
    return pl.pallas_call(
        _mask_kernel,
        grid=grid,
        in_specs=[pl.BlockSpec((_BR, 4), lambda r: (r, 0))],
        out_specs=pl.BlockSpec((_BR, n), lambda r: (r, 0)),
        out_shape=jax.ShapeDtypeStruct((n, n), jnp.bool_),
    )(boxes)


# speedup vs baseline: 144.7376x; 1.9249x over previous
import jax
import jax.numpy as jnp
from jax.experimental import pallas as pl

MIN_WINDOW_SIZE = 33
MAX_WINDOW_SIZE = 99

_BR = 512
_BW = 768


def _mask_kernel(boxes_ref, out_ref):
    r0 = pl.program_id(0) * _BR
    n = out_ref.shape[1]

    wh = boxes_ref[:, 2:4]
    mx = jnp.max(wh, axis=1)
    mn = jnp.min(wh, axis=1)
    scale = jnp.sqrt(mx / mn)
    window = (MIN_WINDOW_SIZE * scale).astype(jnp.int32)
    window = jnp.clip(window, MIN_WINDOW_SIZE, MAX_WINDOW_SIZE)
    half = (window // 2).astype(jnp.int16)
    h = half[:, None]

    out_ref[...] = jnp.zeros(out_ref.shape, jnp.int8)

    ncap = ((n - _BW) // 128) * 128
    start = pl.multiple_of(jnp.clip(r0 - 128, 0, ncap), 128)
    i = r0.astype(jnp.int16) + jax.lax.broadcasted_iota(
        jnp.int16, (_BR, _BW), 0)
    j = start.astype(jnp.int16) + jax.lax.broadcasted_iota(
        jnp.int16, (_BR, _BW), 1)
    band = jnp.where(jnp.abs(j - i) <= h, jnp.int16(1), jnp.int16(0))
    out_ref[:, pl.ds(start, _BW)] = band.astype(jnp.int8)

    ntail = n - (ncap + _BW)
    if ntail > 0:
        jt = jnp.int16(ncap + _BW) + jax.lax.broadcasted_iota(
            jnp.int16, (_BR, ntail), 1)
        it = r0.astype(jnp.int16) + jax.lax.broadcasted_iota(
            jnp.int16, (_BR, ntail), 0)
        tail = jnp.where(jnp.abs(jt - it) <= h, jnp.int16(1), jnp.int16(0))
        out_ref[:, ncap + _BW:] = tail.astype(jnp.int8)


@jax.jit
def kernel(boxes):
    n = boxes.shape[0]
    grid = (pl.cdiv(n, _BR),)
    mask8 = pl.pallas_call(
        _mask_kernel,
        grid=grid,
        in_specs=[pl.BlockSpec((_BR, 4), lambda r: (r, 0))],
        out_specs=pl.BlockSpec((_BR, n), lambda r: (r, 0)),
        out_shape=jax.ShapeDtypeStruct((n, n), jnp.int8),
    )(boxes)
    return mask8.astype(jnp.bool_)


# speedup vs baseline: 145.1172x; 1.0026x over previous
import jax
import jax.numpy as jnp
from jax.experimental import pallas as pl

MIN_WINDOW_SIZE = 33
MAX_WINDOW_SIZE = 99

_BR = 1024
_BW = 1280


def _mask_kernel(boxes_ref, out_ref):
    r0 = pl.program_id(0) * _BR
    n = out_ref.shape[1]

    wh = boxes_ref[:, 2:4]
    mx = jnp.max(wh, axis=1)
    mn = jnp.min(wh, axis=1)
    scale = jnp.sqrt(mx / mn)
    window = (MIN_WINDOW_SIZE * scale).astype(jnp.int32)
    window = jnp.clip(window, MIN_WINDOW_SIZE, MAX_WINDOW_SIZE)
    half = (window // 2).astype(jnp.int16)
    h = half[:, None]

    out_ref[...] = jnp.zeros(out_ref.shape, jnp.int8)

    ncap = ((n - _BW) // 128) * 128
    start = pl.multiple_of(jnp.clip(r0 - 128, 0, ncap), 128)
    i = r0.astype(jnp.int16) + jax.lax.broadcasted_iota(
        jnp.int16, (_BR, _BW), 0)
    j = start.astype(jnp.int16) + jax.lax.broadcasted_iota(
        jnp.int16, (_BR, _BW), 1)
    band = jnp.where(jnp.abs(j - i) <= h, jnp.int16(1), jnp.int16(0))
    out_ref[:, pl.ds(start, _BW)] = band.astype(jnp.int8)

    ntail = n - (ncap + _BW)
    if ntail > 0:
        jt = jnp.int16(ncap + _BW) + jax.lax.broadcasted_iota(
            jnp.int16, (_BR, ntail), 1)
        it = r0.astype(jnp.int16) + jax.lax.broadcasted_iota(
            jnp.int16, (_BR, ntail), 0)
        tail = jnp.where(jnp.abs(jt - it) <= h, jnp.int16(1), jnp.int16(0))
        out_ref[:, ncap + _BW:] = tail.astype(jnp.int8)


@jax.jit
def kernel(boxes):
    n = boxes.shape[0]
    grid = (pl.cdiv(n, _BR),)
    mask8 = pl.pallas_call(
        _mask_kernel,
        grid=grid,
        in_specs=[pl.BlockSpec((_BR, 4), lambda r: (r, 0))],
        out_specs=pl.BlockSpec((_BR, n), lambda r: (r, 0)),
        out_shape=jax.ShapeDtypeStruct((n, n), jnp.int8),
    )(boxes)
    return mask8.astype(jnp.bool_)
